# baseline (speedup 1.0000x reference)
import functools

import jax
import jax.numpy as jnp
from jax import lax
from jax.experimental import pallas as pl
from jax.experimental.pallas import tpu as pltpu
from jax.experimental.pallas import tpu_sc as plsc

B = 16384
D = 32
NC, NS = 2, 16
NW = NC * NS
BPW = B // NW
CHUNK = 128
NCHUNK = BPW // CHUNK

_mesh = plsc.VectorSubcoreMesh(
    core_axis_name="c", subcore_axis_name="s", num_cores=NC, num_subcores=NS
)


@functools.partial(
    pl.kernel,
    out_type=(
        jax.ShapeDtypeStruct((B, D), jnp.float32),
        jax.ShapeDtypeStruct((B, D), jnp.float32),
    ),
    mesh=_mesh,
    scratch_types=(
        pltpu.VMEM((NCHUNK, CHUNK), jnp.int32),
        pltpu.VMEM((NCHUNK, CHUNK), jnp.int32),
        pltpu.VMEM((BPW, D), jnp.float32),
        pltpu.VMEM((BPW, D), jnp.float32),
        pltpu.SemaphoreType.DMA,
    ),
    compiler_params=pltpu.CompilerParams(use_tc_tiling_on_sc=False),
)
def _sc_gather(u_hbm, i_hbm, uemb_hbm, iemb_hbm, uout_hbm, iout_hbm,
               uidx_v, iidx_v, urows_v, irows_v, sem):
    wid = lax.axis_index("s") * NC + lax.axis_index("c")
    base = wid * BPW
    pltpu.sync_copy(u_hbm.at[pl.ds(wid * NCHUNK, NCHUNK)], uidx_v)
    pltpu.sync_copy(i_hbm.at[pl.ds(wid * NCHUNK, NCHUNK)], iidx_v)
    copies = []
    for j in range(NCHUNK):
        copies.append(pltpu.async_copy(
            uemb_hbm.at[uidx_v.at[j]], urows_v.at[pl.ds(j * CHUNK, CHUNK)], sem))
        copies.append(pltpu.async_copy(
            iemb_hbm.at[iidx_v.at[j]], irows_v.at[pl.ds(j * CHUNK, CHUNK)], sem))
    for c in copies:
        c.wait()
    pltpu.sync_copy(urows_v, uout_hbm.at[pl.ds(base, BPW)])
    pltpu.sync_copy(irows_v, iout_hbm.at[pl.ds(base, BPW)])


BT = 1024


def _mlp_body(u_ref, i_ref, w0u_ref, w0i_ref, b0_ref, w1_ref, b1_ref,
              w2_ref, b2_ref, w3_ref, b3_ref, o_ref):
    dot = functools.partial(jnp.dot, preferred_element_type=jnp.float32,
                            precision=lax.Precision.HIGHEST)
    x = jnp.maximum(
        dot(u_ref[...], w0u_ref[...]) + dot(i_ref[...], w0i_ref[...])
        + b0_ref[...], 0.0)
    x = jnp.maximum(dot(x, w1_ref[...]) + b1_ref[...], 0.0)
    x = jnp.maximum(dot(x, w2_ref[...]) + b2_ref[...], 0.0)
    o_ref[...] = dot(x, w3_ref[...]) + b3_ref[...]


def _full(shape):
    return pl.BlockSpec(shape, lambda g: (0, 0))


_mlp_call = pl.pallas_call(
    _mlp_body,
    grid=(B // BT,),
    in_specs=[
        pl.BlockSpec((BT, D), lambda g: (g, 0)),
        pl.BlockSpec((BT, D), lambda g: (g, 0)),
        _full((D, 64)), _full((D, 64)), _full((1, 64)),
        _full((64, 32)), _full((1, 32)),
        _full((32, 16)), _full((1, 16)),
        _full((16, 1)), _full((1, 1)),
    ],
    out_specs=pl.BlockSpec((BT, 1), lambda g: (g, 0)),
    out_shape=jax.ShapeDtypeStruct((B, 1), jnp.float32),
)


def kernel(u, i, user_emb, item_emb, W0, b0, W1, b1, W2, b2, W3, b3):
    u2 = u.astype(jnp.int32).reshape(B // CHUNK, CHUNK)
    i2 = i.astype(jnp.int32).reshape(B // CHUNK, CHUNK)
    u_vec, i_vec = _sc_gather(u2, i2, user_emb, item_emb)
    out2d = _mlp_call(
        u_vec, i_vec,
        W0[:D], W0[D:], b0.reshape(1, -1),
        W1, b1.reshape(1, -1),
        W2, b2.reshape(1, -1),
        W3, b3.reshape(1, -1),
    )
    return out2d.reshape(B)


# speedup vs baseline: 1.0144x; 1.0144x over previous
import functools

import jax
import jax.numpy as jnp
from jax import lax
from jax.experimental import pallas as pl
from jax.experimental.pallas import tpu as pltpu
from jax.experimental.pallas import tpu_sc as plsc

B = 16384
D = 32
PK = 128
RPP = PK // D
NC, NS = 2, 16
NW = NC * NS
BPW = B // NW
CHUNK = 128
NCHUNK = BPW // CHUNK
LANES = 16

_mesh = plsc.VectorSubcoreMesh(
    core_axis_name="c", subcore_axis_name="s", num_cores=NC, num_subcores=NS
)


@functools.partial(
    pl.kernel,
    out_type=(
        jax.ShapeDtypeStruct((B, PK), jnp.float32),
        jax.ShapeDtypeStruct((B, PK), jnp.float32),
    ),
    mesh=_mesh,
    scratch_types=(
        pltpu.VMEM((NCHUNK, CHUNK), jnp.int32),
        pltpu.VMEM((NCHUNK, CHUNK), jnp.int32),
        pltpu.VMEM((BPW, PK), jnp.float32),
        pltpu.SemaphoreType.DMA,
    ),
)
def _sc_gather(u_hbm, i_hbm, utab_hbm, itab_hbm, uout_hbm, iout_hbm,
               uidx_v, iidx_v, rows_v, sem):
    wid = lax.axis_index("s") * NC + lax.axis_index("c")
    base = wid * BPW
    pltpu.sync_copy(u_hbm.at[pl.ds(wid * NCHUNK, NCHUNK)], uidx_v)
    pltpu.sync_copy(i_hbm.at[pl.ds(wid * NCHUNK, NCHUNK)], iidx_v)
    for idx_v in (uidx_v, iidx_v):
        for j in range(NCHUNK):
            for l in range(CHUNK // LANES):
                sl = pl.ds(l * LANES, LANES)
                idx_v[j, sl] = idx_v[j, sl] >> 2
    for idx_v, out_hbm in ((uidx_v, uout_hbm), (iidx_v, iout_hbm)):
        copies = [
            pltpu.async_copy(
                utab_hbm.at[idx_v.at[j]] if out_hbm is uout_hbm
                else itab_hbm.at[idx_v.at[j]],
                rows_v.at[pl.ds(j * CHUNK, CHUNK)], sem)
            for j in range(NCHUNK)
        ]
        for c in copies:
            c.wait()
        pltpu.sync_copy(rows_v, out_hbm.at[pl.ds(base, BPW)])


BT = 1024


def _mlp_body(upk_ref, ipk_ref, u_ref, i_ref, w0u_ref, w0i_ref, b0_ref,
              w1_ref, b1_ref, w2_ref, b2_ref, w3_ref, b3_ref, o_ref):
    dot = functools.partial(jnp.dot, preferred_element_type=jnp.float32)

    def select(pk_ref, idx_ref):
        off = idx_ref[...] & (RPP - 1)
        x = jnp.zeros((BT, D), jnp.float32)
        for k in range(RPP):
            x = jnp.where(off == k, pk_ref[:, k * D:(k + 1) * D], x)
        return x

    xu = select(upk_ref, u_ref)
    xi = select(ipk_ref, i_ref)
    x = jnp.maximum(
        dot(xu, w0u_ref[...]) + dot(xi, w0i_ref[...]) + b0_ref[...], 0.0)
    x = jnp.maximum(dot(x, w1_ref[...]) + b1_ref[...], 0.0)
    x = jnp.maximum(dot(x, w2_ref[...]) + b2_ref[...], 0.0)
    o_ref[...] = dot(x, w3_ref[...]) + b3_ref[...]


def _full(shape):
    return pl.BlockSpec(shape, lambda g: (0, 0))


_mlp_call = pl.pallas_call(
    _mlp_body,
    grid=(B // BT,),
    in_specs=[
        pl.BlockSpec((BT, PK), lambda g: (g, 0)),
        pl.BlockSpec((BT, PK), lambda g: (g, 0)),
        pl.BlockSpec((BT, 1), lambda g: (g, 0)),
        pl.BlockSpec((BT, 1), lambda g: (g, 0)),
        _full((D, 64)), _full((D, 64)), _full((1, 64)),
        _full((64, 32)), _full((1, 32)),
        _full((32, 16)), _full((1, 16)),
        _full((16, 1)), _full((1, 1)),
    ],
    out_specs=pl.BlockSpec((BT, 1), lambda g: (g, 0)),
    out_shape=jax.ShapeDtypeStruct((B, 1), jnp.float32),
)


def kernel(u, i, user_emb, item_emb, W0, b0, W1, b1, W2, b2, W3, b3):
    u32 = u.astype(jnp.int32)
    i32 = i.astype(jnp.int32)
    u2 = u32.reshape(B // CHUNK, CHUNK)
    i2 = i32.reshape(B // CHUNK, CHUNK)
    utab = user_emb.reshape(-1, PK)
    itab = item_emb.reshape(-1, PK)
    upk, ipk = _sc_gather(u2, i2, utab, itab)
    out2d = _mlp_call(
        upk, ipk, u32.reshape(B, 1), i32.reshape(B, 1),
        W0[:D], W0[D:], b0.reshape(1, -1),
        W1, b1.reshape(1, -1),
        W2, b2.reshape(1, -1),
        W3, b3.reshape(1, -1),
    )
    return out2d.reshape(B)
